# Initial kernel scaffold; baseline (speedup 1.0000x reference)
import jax
import jax.numpy as jnp
from jax.experimental import pallas as pl


def kernel(input_ids, output_hidden_states):
    raise NotImplementedError("write your pallas kernel here")


# speedup vs baseline: 3.5268x; 3.5268x over previous
import jax
import jax.numpy as jnp
from jax.experimental import pallas as pl
from jax.experimental.pallas import tpu as pltpu

VOCAB = 1024
BLOCK_ROWS = 512


def _body(ids_ref, logits_ref, hidden_ref):
    ids = ids_ref[...]
    iota = jax.lax.broadcasted_iota(jnp.int32, (BLOCK_ROWS, VOCAB), 1)
    pred = jax.lax.rem(ids + 2, 3) + 1
    logits_ref[...] = jnp.where(iota == pred, 50.0, -50.0)
    hidden_ref[...] = jnp.where(iota == ids, 1.0, 0.0)


def kernel(input_ids, output_hidden_states):
    bsz, seq = input_ids.shape
    n = bsz * seq
    ids2d = input_ids.reshape(n, 1)
    grid = n // BLOCK_ROWS
    out_shape = [
        jax.ShapeDtypeStruct((n, VOCAB), jnp.float32),
        jax.ShapeDtypeStruct((n, VOCAB), jnp.float32),
    ]
    logits, hidden = pl.pallas_call(
        _body,
        grid=(grid,),
        in_specs=[pl.BlockSpec((BLOCK_ROWS, 1), lambda i: (i, 0))],
        out_specs=[
            pl.BlockSpec((BLOCK_ROWS, VOCAB), lambda i: (i, 0)),
            pl.BlockSpec((BLOCK_ROWS, VOCAB), lambda i: (i, 0)),
        ],
        out_shape=out_shape,
        compiler_params=pltpu.CompilerParams(
            dimension_semantics=("arbitrary",),
        ),
    )(ids2d)
    logits = logits.reshape(bsz, seq, VOCAB)
    hidden = hidden.reshape(bsz, seq, VOCAB)
    return (logits, hidden)


# speedup vs baseline: 3.7383x; 1.0599x over previous
import jax
import jax.numpy as jnp
from jax.experimental import pallas as pl
from jax.experimental.pallas import tpu as pltpu

VOCAB = 1024
BLOCK_ROWS = 2048


def _body(ids_ref, logits_ref, hidden_ref):
    ids = ids_ref[...]
    iota = jax.lax.broadcasted_iota(jnp.int32, (BLOCK_ROWS, VOCAB), 1)
    pred = jax.lax.rem(ids + 2, 3) + 1
    logits_ref[...] = jnp.where(iota == pred, 50.0, -50.0)
    hidden_ref[...] = jnp.where(iota == ids, 1.0, 0.0)


def kernel(input_ids, output_hidden_states):
    bsz, seq = input_ids.shape
    n = bsz * seq
    ids2d = input_ids.reshape(n, 1)
    grid = n // BLOCK_ROWS
    out_shape = [
        jax.ShapeDtypeStruct((n, VOCAB), jnp.float32),
        jax.ShapeDtypeStruct((n, VOCAB), jnp.float32),
    ]
    logits, hidden = pl.pallas_call(
        _body,
        grid=(grid,),
        in_specs=[pl.BlockSpec((BLOCK_ROWS, 1), lambda i: (i, 0))],
        out_specs=[
            pl.BlockSpec((BLOCK_ROWS, VOCAB), lambda i: (i, 0)),
            pl.BlockSpec((BLOCK_ROWS, VOCAB), lambda i: (i, 0)),
        ],
        out_shape=out_shape,
        compiler_params=pltpu.CompilerParams(
            dimension_semantics=("arbitrary",),
        ),
    )(ids2d)
    logits = logits.reshape(bsz, seq, VOCAB)
    hidden = hidden.reshape(bsz, seq, VOCAB)
    return (logits, hidden)


# speedup vs baseline: 3.7511x; 1.0034x over previous
import jax
import jax.numpy as jnp
from jax.experimental import pallas as pl
from jax.experimental.pallas import tpu as pltpu

VOCAB = 1024
BLOCK_ROWS = 1024


def _body(ids_ref, logits_ref, hidden_ref):
    ids = ids_ref[...]
    iota = jax.lax.broadcasted_iota(jnp.int32, (BLOCK_ROWS, VOCAB), 1)
    pred = jax.lax.rem(ids + 2, 3) + 1
    logits_ref[...] = jnp.where(iota == pred, 50.0, -50.0)
    hidden_ref[...] = jnp.where(iota == ids, 1.0, 0.0)


def kernel(input_ids, output_hidden_states):
    bsz, seq = input_ids.shape
    n = bsz * seq
    ids2d = input_ids.reshape(n, 1)
    grid = n // BLOCK_ROWS
    out_shape = [
        jax.ShapeDtypeStruct((n, VOCAB), jnp.float32),
        jax.ShapeDtypeStruct((n, VOCAB), jnp.float32),
    ]
    logits, hidden = pl.pallas_call(
        _body,
        grid=(grid,),
        in_specs=[pl.BlockSpec((BLOCK_ROWS, 1), lambda i: (i, 0))],
        out_specs=[
            pl.BlockSpec((BLOCK_ROWS, VOCAB), lambda i: (i, 0)),
            pl.BlockSpec((BLOCK_ROWS, VOCAB), lambda i: (i, 0)),
        ],
        out_shape=out_shape,
        compiler_params=pltpu.CompilerParams(
            dimension_semantics=("arbitrary",),
        ),
    )(ids2d)
    logits = logits.reshape(bsz, seq, VOCAB)
    hidden = hidden.reshape(bsz, seq, VOCAB)
    return (logits, hidden)
